# baseline (speedup 1.0000x reference)
import functools

import jax
import jax.numpy as jnp
from jax import lax
from jax.experimental import pallas as pl
from jax.experimental.pallas import tpu as pltpu
from jax.experimental.pallas import tpu_sc as plsc

NUM_NODES = 50000
B = 4
N_PER = 8192
D = 128
C = 128

NC = 2
NS = 16
NW = NC * NS

POS = B * N_PER
HPOS = POS // 2
PA = HPOS // NW
CHUNK = 16
NCHUNK = D // CHUNK
SEG = N_PER // NS
ROWS_PER_TILE = NUM_NODES // NS


def _sc_body(half, x_hbm, basex_hbm, idxr_hbm, bg_hbm, sg_hbm,
             acc, idx_a, idx_all, bbuf, xbuf, gbuf, zbuf,
             sem, sem2, sem3):
    c = lax.axis_index("c")
    s = lax.axis_index("s")
    wid = c * NS + s
    t = 2 * half + c

    def _z(i, _):
        zbuf[i, pl.ds(0, 16)] = jnp.zeros((16,), jnp.float32)
        return 0
    lax.fori_loop(0, 128, _z, 0)

    full, rem = divmod(ROWS_PER_TILE, 128)
    for k in range(full):
        pltpu.sync_copy(zbuf, acc.at[pl.ds(s * ROWS_PER_TILE + k * 128, 128)])
    if rem:
        pltpu.sync_copy(zbuf.at[pl.ds(0, rem)],
                        acc.at[pl.ds(s * ROWS_PER_TILE + full * 128, rem)])

    nr = N_PER // 128
    pltpu.sync_copy(idxr_hbm.at[pl.ds(half * (HPOS // 128) + 4 * wid, 4)],
                    idx_a)
    pltpu.sync_copy(idxr_hbm.at[pl.ds(t * nr + 4 * s, 4)], idx_all)

    plsc.subcore_barrier()

    def _phase_b(i, _):
        rowbase = t * N_PER + SEG * s
        outbase = c * N_PER + SEG * s
        colbase = CHUNK * i
        l1 = pltpu.async_copy(
            x_hbm.at[pl.ds(rowbase, SEG), pl.ds(colbase, CHUNK)], xbuf, sem3)
        l1.wait()
        plsc.subcore_barrier()
        sc = [pltpu.async_copy(xbuf.at[pl.ds(128 * j, 128)],
                               acc.at[idx_all.at[j]], sem, add=True)
              for j in range(4)]
        for d in sc:
            d.wait()
        plsc.subcore_barrier()
        ga = [pltpu.async_copy(acc.at[idx_all.at[j]],
                               gbuf.at[pl.ds(128 * j, 128)], sem)
              for j in range(4)]
        for d in ga:
            d.wait()
        plsc.subcore_barrier()
        zs = [pltpu.async_copy(zbuf, acc.at[idx_all.at[j]], sem)
              for j in range(4)]
        ow = pltpu.async_copy(
            gbuf, sg_hbm.at[pl.ds(outbase, SEG), pl.ds(colbase, CHUNK)], sem2)

        @pl.when(i < 4)
        def _pa():
            g0 = pltpu.async_copy(basex_hbm.at[idx_a.at[i]], bbuf, sem3)
            g0.wait()
            w0 = pltpu.async_copy(
                bbuf, bg_hbm.at[pl.ds(PA * wid + 128 * i, 128)], sem3)
            w0.wait()

        for d in zs:
            d.wait()
        ow.wait()
        return 0
    lax.fori_loop(0, NCHUNK, _phase_b, 0)


def _sc_half(x, base_x, idxr, half):
    mesh = plsc.VectorSubcoreMesh(core_axis_name="c", subcore_axis_name="s",
                                  num_cores=NC, num_subcores=NS)
    f = pl.kernel(
        functools.partial(_sc_body, half),
        out_type=(jax.ShapeDtypeStruct((HPOS, D), jnp.float32),
                  jax.ShapeDtypeStruct((HPOS, D), jnp.float32)),
        mesh=mesh,
        scratch_types=[
            pltpu.VMEM_SHARED((NUM_NODES, CHUNK), jnp.float32),
            pltpu.VMEM((4, 128), jnp.int32),
            pltpu.VMEM((4, 128), jnp.int32),
            pltpu.VMEM((128, D), jnp.float32),
            pltpu.VMEM((SEG, CHUNK), jnp.float32),
            pltpu.VMEM((SEG, CHUNK), jnp.float32),
            pltpu.VMEM((128, CHUNK), jnp.float32),
            pltpu.SemaphoreType.DMA,
            pltpu.SemaphoreType.DMA,
            pltpu.SemaphoreType.DMA,
        ],
        compiler_params=pltpu.CompilerParams(use_tc_tiling_on_sc=False),
        name=f"sc_half{half}",
    )
    return f(x, base_x, idxr)


def _gelu(h):
    return 0.5 * h * (1.0 + lax.erf(h * (2.0 ** -0.5)))


def _ln(h, g, b):
    m = jnp.mean(h, axis=-1, keepdims=True)
    v = jnp.mean((h - m) ** 2, axis=-1, keepdims=True)
    return (h - m) * lax.rsqrt(v + 1e-5) * g + b


def _tc_body(bg_ref, sg_ref, w1d_ref, b1d_ref, w2d_ref, b2d_ref,
             lndg_ref, lndb_ref, lnug_ref, lnub_ref,
             w1u_ref, b1u_ref, w2u_ref, b2u_ref, out_ref):
    def bdot(a, w):
        return jnp.dot(a, w, preferred_element_type=jnp.float32)

    inp = bg_ref[...] + sg_ref[...]
    h = _ln(inp, lndg_ref[...], lndb_ref[...])
    h = _gelu(bdot(h, w1d_ref[...]) + b1d_ref[...])
    h = bdot(h, w2d_ref[...]) + b2d_ref[...]
    h = _ln(h, lnug_ref[...], lnub_ref[...])
    h = _gelu(bdot(h, w1u_ref[...]) + b1u_ref[...])
    out_ref[...] = bdot(h, w2u_ref[...]) + b2u_ref[...]


def _tc_mlp(bg, sg, W1d, b1d, W2d, b2d, ln_d_g, ln_d_b,
            ln_u_g, ln_u_b, W1u, b1u, W2u, b2u):
    R = 1024
    n = bg.shape[0]
    grid = (n // R,)
    row_spec = pl.BlockSpec((R, D), lambda i: (i, 0))

    def rep(shape):
        return pl.BlockSpec(shape, lambda i: tuple(0 for _ in shape))

    return pl.pallas_call(
        _tc_body,
        grid=grid,
        in_specs=[
            row_spec, row_spec,
            rep((D, 2 * D)), rep((1, 2 * D)), rep((2 * D, C)), rep((1, C)),
            rep((1, D)), rep((1, D)), rep((1, C)), rep((1, C)),
            rep((C, 2 * C)), rep((1, 2 * C)), rep((2 * C, D)), rep((1, D)),
        ],
        out_specs=row_spec,
        out_shape=jax.ShapeDtypeStruct((n, D), jnp.float32),
        compiler_params=pltpu.CompilerParams(skip_device_barrier=True),
    )(bg, sg, W1d, b1d.reshape(1, -1), W2d, b2d.reshape(1, -1),
      ln_d_g.reshape(1, -1), ln_d_b.reshape(1, -1),
      ln_u_g.reshape(1, -1), ln_u_b.reshape(1, -1),
      W1u, b1u.reshape(1, -1), W2u, b2u.reshape(1, -1))


def kernel(x, base_x, ln_d_g, ln_d_b, W1d, b1d, W2d, b2d,
           ln_u_g, ln_u_b, W1u, b1u, W2u, b2u, indices_subnodes):
    idxr = indices_subnodes.reshape(POS // 128, 128).astype(jnp.int32)
    mlp = functools.partial(
        _tc_mlp, W1d=W1d, b1d=b1d, W2d=W2d, b2d=b2d, ln_d_g=ln_d_g,
        ln_d_b=ln_d_b, ln_u_g=ln_u_g, ln_u_b=ln_u_b, W1u=W1u, b1u=b1u,
        W2u=W2u, b2u=b2u)
    bg0, sg0 = _sc_half(x, base_x, idxr, 0)
    bg1, sg1 = _sc_half(x, base_x, idxr, 1)
    o0 = mlp(bg0, sg0)
    o1 = mlp(bg1, sg1)
    return jnp.concatenate([o0, o1], axis=0)


# speedup vs baseline: 1.0823x; 1.0823x over previous
import functools

import jax
import jax.numpy as jnp
from jax import lax
from jax.experimental import pallas as pl
from jax.experimental.pallas import tpu as pltpu
from jax.experimental.pallas import tpu_sc as plsc

NUM_NODES = 50000
B = 4
N_PER = 8192
D = 128
C = 128

NC = 2
NS = 16
NW = NC * NS

POS = B * N_PER
PA = POS // NW
CHUNK = 16
NCHUNK = D // CHUNK
NTASK = (B // NC) * NCHUNK
SEG = N_PER // NS
ROWS_PER_TILE = NUM_NODES // NS


def _sc_body(x_hbm, basex_hbm, idxr_hbm, bg_hbm, sg_hbm,
             acc0, acc1, idx_a, idx_all, bbuf, xbuf, gbuf, zbuf,
             sem, sem2, sem3):
    c = lax.axis_index("c")
    s = lax.axis_index("s")
    wid = c * NS + s

    def _z(i, _):
        zbuf[i, pl.ds(0, 16)] = jnp.zeros((16,), jnp.float32)
        return 0
    lax.fori_loop(0, 128, _z, 0)

    full, rem = divmod(ROWS_PER_TILE, 128)
    for acc in (acc0, acc1):
        for k in range(full):
            pltpu.sync_copy(zbuf,
                            acc.at[pl.ds(s * ROWS_PER_TILE + k * 128, 128)])
        if rem:
            pltpu.sync_copy(zbuf.at[pl.ds(0, rem)],
                            acc.at[pl.ds(s * ROWS_PER_TILE + full * 128, rem)])

    nr = N_PER // 128
    pltpu.sync_copy(idxr_hbm.at[pl.ds(8 * wid, 8)], idx_a)
    pltpu.sync_copy(idxr_hbm.at[pl.ds(2 * c * nr + 4 * s, 4)],
                    idx_all.at[pl.ds(0, 4)])
    pltpu.sync_copy(idxr_hbm.at[pl.ds((2 * c + 1) * nr + 4 * s, 4)],
                    idx_all.at[pl.ds(4, 4)])

    def xsrc(k):
        t = 2 * c + k // NCHUNK
        return x_hbm.at[pl.ds(t * N_PER + SEG * s, SEG),
                        pl.ds(CHUNK * (k % NCHUNK), CHUNK)]

    def sgdst(k):
        t = 2 * c + k // NCHUNK
        return sg_hbm.at[pl.ds(t * N_PER + SEG * s, SEG),
                         pl.ds(CHUNK * (k % NCHUNK), CHUNK)]

    pltpu.sync_copy(xsrc(0), xbuf)
    plsc.subcore_barrier()

    def _task(k, acc, first, last, do_pa, i):
        idxrow = lambda kk, j: idx_all.at[4 * (kk // NCHUNK) + j]
        sc = [pltpu.async_copy(xbuf.at[pl.ds(128 * j, 128)],
                               acc.at[idxrow(k, j)], sem, add=True)
              for j in range(4)]
        if not first:
            oacc = acc1 if acc is acc0 else acc0
            zs = [pltpu.async_copy(zbuf, oacc.at[idxrow(k - 1, j)], sem2)
                  for j in range(4)]
            ow = pltpu.async_copy(gbuf, sgdst(k - 1), sem3)
            for d in zs:
                d.wait()
            ow.wait()
        for d in sc:
            d.wait()
        plsc.subcore_barrier()
        ga = [pltpu.async_copy(acc.at[idxrow(k, j)],
                               gbuf.at[pl.ds(128 * j, 128)], sem)
              for j in range(4)]
        if not last:
            lx = pltpu.async_copy(xsrc(k + 1), xbuf, sem3)
        if do_pa:
            for h in range(2):
                g0 = pltpu.async_copy(
                    basex_hbm.at[idx_a.at[i, pl.ds(64 * h, 64)]], bbuf, sem2)
                g0.wait()
                w0 = pltpu.async_copy(
                    bbuf, bg_hbm.at[pl.ds(PA * wid + 128 * i + 64 * h, 64)],
                    sem2)
                w0.wait()
        for d in ga:
            d.wait()
        if not last:
            lx.wait()
        plsc.subcore_barrier()

    _task(0, acc0, True, False, False, 0)

    def _mid(i, _):
        _task(2 * i + 1, acc1, False, False, True, i)
        _task(2 * i + 2, acc0, False, False, False, i)
        return 0
    lax.fori_loop(0, (NTASK - 2) // 2, _mid, 0)

    _task(NTASK - 1, acc1, False, True, True, (NTASK - 2) // 2)
    ow = pltpu.async_copy(gbuf, sgdst(NTASK - 1), sem3)
    ow.wait()


def _sc_gather_scatter(x, base_x, idx_flat):
    idxr = idx_flat.reshape(POS // 128, 128)
    mesh = plsc.VectorSubcoreMesh(core_axis_name="c", subcore_axis_name="s",
                                  num_cores=NC, num_subcores=NS)
    f = pl.kernel(
        _sc_body,
        out_type=(jax.ShapeDtypeStruct((POS, D), jnp.float32),
                  jax.ShapeDtypeStruct((POS, D), jnp.float32)),
        mesh=mesh,
        scratch_types=[
            pltpu.VMEM_SHARED((NUM_NODES, CHUNK), jnp.float32),
            pltpu.VMEM_SHARED((NUM_NODES, CHUNK), jnp.float32),
            pltpu.VMEM((8, 128), jnp.int32),
            pltpu.VMEM((8, 128), jnp.int32),
            pltpu.VMEM((64, D), jnp.float32),
            pltpu.VMEM((SEG, CHUNK), jnp.float32),
            pltpu.VMEM((SEG, CHUNK), jnp.float32),
            pltpu.VMEM((128, CHUNK), jnp.float32),
            pltpu.SemaphoreType.DMA,
            pltpu.SemaphoreType.DMA,
            pltpu.SemaphoreType.DMA,
        ],
        compiler_params=pltpu.CompilerParams(use_tc_tiling_on_sc=False),
    )
    return f(x, base_x, idxr)


def _gelu(h):
    return 0.5 * h * (1.0 + lax.erf(h * (2.0 ** -0.5)))


def _ln(h, g, b):
    m = jnp.mean(h, axis=-1, keepdims=True)
    v = jnp.mean((h - m) ** 2, axis=-1, keepdims=True)
    return (h - m) * lax.rsqrt(v + 1e-5) * g + b


def _tc_body(bg_ref, sg_ref, w1d_ref, b1d_ref, w2d_ref, b2d_ref,
             lndg_ref, lndb_ref, lnug_ref, lnub_ref,
             w1u_ref, b1u_ref, w2u_ref, b2u_ref, out_ref):
    def bdot(a, w):
        return jnp.dot(a, w, preferred_element_type=jnp.float32)

    inp = bg_ref[...] + sg_ref[...]
    h = _ln(inp, lndg_ref[...], lndb_ref[...])
    h = _gelu(bdot(h, w1d_ref[...]) + b1d_ref[...])
    h = bdot(h, w2d_ref[...]) + b2d_ref[...]
    h = _ln(h, lnug_ref[...], lnub_ref[...])
    h = _gelu(bdot(h, w1u_ref[...]) + b1u_ref[...])
    out_ref[...] = bdot(h, w2u_ref[...]) + b2u_ref[...]


def _tc_mlp(bg, sg, W1d, b1d, W2d, b2d, ln_d_g, ln_d_b,
            ln_u_g, ln_u_b, W1u, b1u, W2u, b2u):
    R = 1024
    n = bg.shape[0]
    grid = (n // R,)
    row_spec = pl.BlockSpec((R, D), lambda i: (i, 0))

    def rep(shape):
        return pl.BlockSpec(shape, lambda i: tuple(0 for _ in shape))

    return pl.pallas_call(
        _tc_body,
        grid=grid,
        in_specs=[
            row_spec, row_spec,
            rep((D, 2 * D)), rep((1, 2 * D)), rep((2 * D, C)), rep((1, C)),
            rep((1, D)), rep((1, D)), rep((1, C)), rep((1, C)),
            rep((C, 2 * C)), rep((1, 2 * C)), rep((2 * C, D)), rep((1, D)),
        ],
        out_specs=row_spec,
        out_shape=jax.ShapeDtypeStruct((n, D), jnp.float32),
    )(bg, sg, W1d, b1d.reshape(1, -1), W2d, b2d.reshape(1, -1),
      ln_d_g.reshape(1, -1), ln_d_b.reshape(1, -1),
      ln_u_g.reshape(1, -1), ln_u_b.reshape(1, -1),
      W1u, b1u.reshape(1, -1), W2u, b2u.reshape(1, -1))


def kernel(x, base_x, ln_d_g, ln_d_b, W1d, b1d, W2d, b2d,
           ln_u_g, ln_u_b, W1u, b1u, W2u, b2u, indices_subnodes):
    idx_flat = indices_subnodes.reshape(POS).astype(jnp.int32)
    bg, sg = _sc_gather_scatter(x, base_x, idx_flat)
    return _tc_mlp(bg, sg, W1d, b1d, W2d, b2d, ln_d_g, ln_d_b,
                   ln_u_g, ln_u_b, W1u, b1u, W2u, b2u)


# speedup vs baseline: 1.1840x; 1.0939x over previous
import functools

import jax
import jax.numpy as jnp
from jax import lax
from jax.experimental import pallas as pl
from jax.experimental.pallas import tpu as pltpu
from jax.experimental.pallas import tpu_sc as plsc

NUM_NODES = 50000
B = 4
N_PER = 8192
D = 128
C = 128

NC = 2
NS = 16
NW = NC * NS

POS = B * N_PER
PA = POS // NW
CHUNK = 16
NCHUNK = D // CHUNK
NTASK = (B // NC) * NCHUNK
SEG = N_PER // NS
ROWS_PER_TILE = NUM_NODES // NS


def _sc_body(x_hbm, basex_hbm, idxr_hbm, bg_hbm, sg_hbm,
             acc0, acc1, idx_a, idx_all, bbuf, xbuf, gbuf, zbuf,
             sem, sem2, sem3):
    c = lax.axis_index("c")
    s = lax.axis_index("s")
    wid = c * NS + s

    def _z(i, _):
        zbuf[i, pl.ds(0, 16)] = jnp.zeros((16,), jnp.float32)
        return 0
    lax.fori_loop(0, 128, _z, 0)

    full, rem = divmod(ROWS_PER_TILE, 128)
    for acc in (acc0, acc1):
        for k in range(full):
            pltpu.sync_copy(zbuf,
                            acc.at[pl.ds(s * ROWS_PER_TILE + k * 128, 128)])
        if rem:
            pltpu.sync_copy(zbuf.at[pl.ds(0, rem)],
                            acc.at[pl.ds(s * ROWS_PER_TILE + full * 128, rem)])

    nr = N_PER // 128
    pltpu.sync_copy(idxr_hbm.at[pl.ds(8 * wid, 8)], idx_a)
    pltpu.sync_copy(idxr_hbm.at[pl.ds(2 * c * nr + 4 * s, 4)],
                    idx_all.at[pl.ds(0, 4)])
    pltpu.sync_copy(idxr_hbm.at[pl.ds((2 * c + 1) * nr + 4 * s, 4)],
                    idx_all.at[pl.ds(4, 4)])

    def xsrc(k):
        t = 2 * c + k // NCHUNK
        return x_hbm.at[pl.ds(t * N_PER + SEG * s, SEG),
                        pl.ds(CHUNK * (k % NCHUNK), CHUNK)]

    def sgdst(k):
        t = 2 * c + k // NCHUNK
        return sg_hbm.at[pl.ds(t * N_PER + SEG * s, SEG),
                         pl.ds(CHUNK * (k % NCHUNK), CHUNK)]

    pltpu.sync_copy(xsrc(0), xbuf)
    plsc.subcore_barrier()

    def _task(k, acc, first, last, do_pa, i):
        idxrow = lambda kk, j: idx_all.at[4 * (kk // NCHUNK) + j]
        sc = [pltpu.async_copy(xbuf.at[pl.ds(128 * j, 128)],
                               acc.at[idxrow(k, j)], sem, add=True)
              for j in range(4)]
        if not first:
            oacc = acc1 if acc is acc0 else acc0
            zs = [pltpu.async_copy(zbuf, oacc.at[idxrow(k - 1, j)], sem2)
                  for j in range(4)]
            ow = pltpu.async_copy(gbuf, sgdst(k - 1), sem3)
            for d in zs:
                d.wait()
            ow.wait()
        for d in sc:
            d.wait()
        plsc.subcore_barrier()
        ga = [pltpu.async_copy(acc.at[idxrow(k, j)],
                               gbuf.at[pl.ds(128 * j, 128)], sem)
              for j in range(4)]
        if not last:
            lx = pltpu.async_copy(xsrc(k + 1), xbuf, sem3)
        if do_pa:
            for h in range(2):
                g0 = pltpu.async_copy(
                    basex_hbm.at[idx_a.at[i, pl.ds(64 * h, 64)]], bbuf, sem2)
                g0.wait()
                w0 = pltpu.async_copy(
                    bbuf, bg_hbm.at[pl.ds(PA * wid + 128 * i + 64 * h, 64)],
                    sem2)
                w0.wait()
        for d in ga:
            d.wait()
        if not last:
            lx.wait()
        plsc.subcore_barrier()

    _task(0, acc0, True, False, False, 0)

    def _mid(i, _):
        _task(2 * i + 1, acc1, False, False, True, i)
        _task(2 * i + 2, acc0, False, False, False, i)
        return 0
    lax.fori_loop(0, (NTASK - 2) // 2, _mid, 0)

    _task(NTASK - 1, acc1, False, True, True, (NTASK - 2) // 2)
    ow = pltpu.async_copy(gbuf, sgdst(NTASK - 1), sem3)
    ow.wait()


def _sc_gather_scatter(x, base_x, idx_flat):
    idxr = idx_flat.reshape(POS // 128, 128)
    mesh = plsc.VectorSubcoreMesh(core_axis_name="c", subcore_axis_name="s",
                                  num_cores=NC, num_subcores=NS)
    f = pl.kernel(
        _sc_body,
        out_type=(jax.ShapeDtypeStruct((POS, D), jnp.float32),
                  jax.ShapeDtypeStruct((POS, D), jnp.float32)),
        mesh=mesh,
        scratch_types=[
            pltpu.VMEM_SHARED((NUM_NODES, CHUNK), jnp.float32),
            pltpu.VMEM_SHARED((NUM_NODES, CHUNK), jnp.float32),
            pltpu.VMEM((8, 128), jnp.int32),
            pltpu.VMEM((8, 128), jnp.int32),
            pltpu.VMEM((64, D), jnp.float32),
            pltpu.VMEM((SEG, CHUNK), jnp.float32),
            pltpu.VMEM((SEG, CHUNK), jnp.float32),
            pltpu.VMEM((128, CHUNK), jnp.float32),
            pltpu.SemaphoreType.DMA,
            pltpu.SemaphoreType.DMA,
            pltpu.SemaphoreType.DMA,
        ],
        compiler_params=pltpu.CompilerParams(use_tc_tiling_on_sc=False),
    )
    return f(x, base_x, idxr)


def _gelu(h):
    return 0.5 * h * (1.0 + lax.erf(h * (2.0 ** -0.5)))


def _ln(h, g, b):
    m = jnp.mean(h, axis=-1, keepdims=True)
    v = jnp.mean((h - m) ** 2, axis=-1, keepdims=True)
    return (h - m) * lax.rsqrt(v + 1e-5) * g + b


def _tc_body(bg_ref, sg_ref, w1d_ref, b1d_ref, w2d_ref, b2d_ref,
             lndg_ref, lndb_ref, lnug_ref, lnub_ref,
             w1u_ref, b1u_ref, w2u_ref, b2u_ref, out_ref):
    def bdot(a, w):
        return jnp.dot(a, w, preferred_element_type=jnp.float32)

    inp = bg_ref[...] + sg_ref[...]
    h = _ln(inp, lndg_ref[...], lndb_ref[...])
    h = _gelu(bdot(h, w1d_ref[...]) + b1d_ref[...])
    h = bdot(h, w2d_ref[...]) + b2d_ref[...]
    h = _ln(h, lnug_ref[...], lnub_ref[...])
    h = _gelu(bdot(h, w1u_ref[...]) + b1u_ref[...])
    out_ref[...] = bdot(h, w2u_ref[...]) + b2u_ref[...]


def _tc_mlp(bg, sg, W1d, b1d, W2d, b2d, ln_d_g, ln_d_b,
            ln_u_g, ln_u_b, W1u, b1u, W2u, b2u):
    R = 2048
    n = bg.shape[0]
    grid = (n // R,)
    row_spec = pl.BlockSpec((R, D), lambda i: (i, 0))

    def rep(shape):
        return pl.BlockSpec(shape, lambda i: tuple(0 for _ in shape))

    return pl.pallas_call(
        _tc_body,
        grid=grid,
        in_specs=[
            row_spec, row_spec,
            rep((D, 2 * D)), rep((1, 2 * D)), rep((2 * D, C)), rep((1, C)),
            rep((1, D)), rep((1, D)), rep((1, C)), rep((1, C)),
            rep((C, 2 * C)), rep((1, 2 * C)), rep((2 * C, D)), rep((1, D)),
        ],
        out_specs=row_spec,
        out_shape=jax.ShapeDtypeStruct((n, D), jnp.float32),
    )(bg, sg, W1d, b1d.reshape(1, -1), W2d, b2d.reshape(1, -1),
      ln_d_g.reshape(1, -1), ln_d_b.reshape(1, -1),
      ln_u_g.reshape(1, -1), ln_u_b.reshape(1, -1),
      W1u, b1u.reshape(1, -1), W2u, b2u.reshape(1, -1))


def kernel(x, base_x, ln_d_g, ln_d_b, W1d, b1d, W2d, b2d,
           ln_u_g, ln_u_b, W1u, b1u, W2u, b2u, indices_subnodes):
    idx_flat = indices_subnodes.reshape(POS).astype(jnp.int32)
    bg, sg = _sc_gather_scatter(x, base_x, idx_flat)
    return _tc_mlp(bg, sg, W1d, b1d, W2d, b2d, ln_d_g, ln_d_b,
                   ln_u_g, ln_u_b, W1u, b1u, W2u, b2u)


# speedup vs baseline: 1.2151x; 1.0263x over previous
import functools

import jax
import jax.numpy as jnp
from jax import lax
from jax.experimental import pallas as pl
from jax.experimental.pallas import tpu as pltpu
from jax.experimental.pallas import tpu_sc as plsc

NUM_NODES = 50000
B = 4
N_PER = 8192
D = 128
C = 128

NC = 2
NS = 16
NW = NC * NS

POS = B * N_PER
PA = POS // NW
CHUNK = 16
NCHUNK = D // CHUNK
NTASK = (B // NC) * NCHUNK
SEG = N_PER // NS
ROWS_PER_TILE = NUM_NODES // NS


def _sc_body(x_hbm, basex_hbm, idxr_hbm, bg_hbm, sg_hbm,
             acc0, acc1, idx_a, idx_all, bbuf, xbuf, gbuf, zbuf,
             sem, sem2, sem3):
    c = lax.axis_index("c")
    s = lax.axis_index("s")
    wid = c * NS + s

    def _z(i, _):
        zbuf[i, pl.ds(0, 16)] = jnp.zeros((16,), jnp.float32)
        return 0
    lax.fori_loop(0, 128, _z, 0)

    full, rem = divmod(ROWS_PER_TILE, 128)
    for acc in (acc0, acc1):
        for k in range(full):
            pltpu.sync_copy(zbuf,
                            acc.at[pl.ds(s * ROWS_PER_TILE + k * 128, 128)])
        if rem:
            pltpu.sync_copy(zbuf.at[pl.ds(0, rem)],
                            acc.at[pl.ds(s * ROWS_PER_TILE + full * 128, rem)])

    nr = N_PER // 128
    pltpu.sync_copy(idxr_hbm.at[pl.ds(8 * wid, 8)], idx_a)
    pltpu.sync_copy(idxr_hbm.at[pl.ds(2 * c * nr + 4 * s, 4)],
                    idx_all.at[pl.ds(0, 4)])
    pltpu.sync_copy(idxr_hbm.at[pl.ds((2 * c + 1) * nr + 4 * s, 4)],
                    idx_all.at[pl.ds(4, 4)])

    def xsrc(k):
        t = 2 * c + k // NCHUNK
        return x_hbm.at[pl.ds(t * N_PER + SEG * s, SEG),
                        pl.ds(CHUNK * (k % NCHUNK), CHUNK)]

    def sgdst(k):
        t = 2 * c + k // NCHUNK
        return sg_hbm.at[pl.ds(t * N_PER + SEG * s, SEG),
                         pl.ds(CHUNK * (k % NCHUNK), CHUNK)]

    pltpu.sync_copy(xsrc(0), xbuf)
    plsc.subcore_barrier()

    def _task(k, acc, first, last, do_pa, i):
        idxrow = lambda kk, j: idx_all.at[4 * (kk // NCHUNK) + j]
        sc = [pltpu.async_copy(xbuf.at[pl.ds(128 * j, 128)],
                               acc.at[idxrow(k, j)], sem, add=True)
              for j in range(4)]
        if not first:
            oacc = acc1 if acc is acc0 else acc0
            zs = [pltpu.async_copy(zbuf, oacc.at[idxrow(k - 1, j)], sem2)
                  for j in range(4)]
            ow = pltpu.async_copy(gbuf, sgdst(k - 1), sem3)
            for d in zs:
                d.wait()
            ow.wait()
        for d in sc:
            d.wait()
        plsc.subcore_barrier()
        ga = [pltpu.async_copy(acc.at[idxrow(k, j)],
                               gbuf.at[pl.ds(128 * j, 128)], sem)
              for j in range(4)]
        if not last:
            lx = pltpu.async_copy(xsrc(k + 1), xbuf, sem3)
        if do_pa:
            for h in range(2):
                g0 = pltpu.async_copy(
                    basex_hbm.at[idx_a.at[i, pl.ds(64 * h, 64)]], bbuf, sem2)
                g0.wait()
                w0 = pltpu.async_copy(
                    bbuf, bg_hbm.at[pl.ds(PA * wid + 128 * i + 64 * h, 64)],
                    sem2)
                w0.wait()
        for d in ga:
            d.wait()
        if not last:
            lx.wait()
        plsc.subcore_barrier()

    _task(0, acc0, True, False, False, 0)

    def _mid(i, _):
        _task(2 * i + 1, acc1, False, False, True, i)
        _task(2 * i + 2, acc0, False, False, False, i)
        return 0
    lax.fori_loop(0, (NTASK - 2) // 2, _mid, 0)

    _task(NTASK - 1, acc1, False, True, True, (NTASK - 2) // 2)
    ow = pltpu.async_copy(gbuf, sgdst(NTASK - 1), sem3)
    ow.wait()


def _sc_gather_scatter(x, base_x, idx_flat):
    idxr = idx_flat.reshape(POS // 128, 128)
    mesh = plsc.VectorSubcoreMesh(core_axis_name="c", subcore_axis_name="s",
                                  num_cores=NC, num_subcores=NS)
    f = pl.kernel(
        _sc_body,
        out_type=(jax.ShapeDtypeStruct((POS, D), jnp.float32),
                  jax.ShapeDtypeStruct((POS, D), jnp.float32)),
        mesh=mesh,
        scratch_types=[
            pltpu.VMEM_SHARED((NUM_NODES, CHUNK), jnp.float32),
            pltpu.VMEM_SHARED((NUM_NODES, CHUNK), jnp.float32),
            pltpu.VMEM((8, 128), jnp.int32),
            pltpu.VMEM((8, 128), jnp.int32),
            pltpu.VMEM((64, D), jnp.float32),
            pltpu.VMEM((SEG, CHUNK), jnp.float32),
            pltpu.VMEM((SEG, CHUNK), jnp.float32),
            pltpu.VMEM((128, CHUNK), jnp.float32),
            pltpu.SemaphoreType.DMA,
            pltpu.SemaphoreType.DMA,
            pltpu.SemaphoreType.DMA,
        ],
        compiler_params=pltpu.CompilerParams(use_tc_tiling_on_sc=False),
    )
    return f(x, base_x, idxr)


def _gelu(h):
    return 0.5 * h * (1.0 + lax.erf(h * (2.0 ** -0.5)))


def _ln(h, g, b):
    m = jnp.mean(h, axis=-1, keepdims=True)
    v = jnp.mean((h - m) ** 2, axis=-1, keepdims=True)
    return (h - m) * lax.rsqrt(v + 1e-5) * g + b


def _tc_body(bg_ref, sg_ref, w1d_ref, b1d_ref, w2d_ref, b2d_ref,
             lndg_ref, lndb_ref, lnug_ref, lnub_ref,
             w1u_ref, b1u_ref, w2u_ref, b2u_ref, out_ref):
    def bdot(a, w):
        return jnp.dot(a, w, preferred_element_type=jnp.float32)

    inp = bg_ref[...] + sg_ref[...]
    h = _ln(inp, lndg_ref[...], lndb_ref[...])
    h = _gelu(bdot(h, w1d_ref[...]) + b1d_ref[...])
    h = bdot(h, w2d_ref[...]) + b2d_ref[...]
    h = _ln(h, lnug_ref[...], lnub_ref[...])
    h = _gelu(bdot(h, w1u_ref[...]) + b1u_ref[...])
    out_ref[...] = bdot(h, w2u_ref[...]) + b2u_ref[...]


def _tc_mlp(bg, sg, W1d, b1d, W2d, b2d, ln_d_g, ln_d_b,
            ln_u_g, ln_u_b, W1u, b1u, W2u, b2u):
    R = 4096
    n = bg.shape[0]
    grid = (n // R,)
    row_spec = pl.BlockSpec((R, D), lambda i: (i, 0))

    def rep(shape):
        return pl.BlockSpec(shape, lambda i: tuple(0 for _ in shape))

    return pl.pallas_call(
        _tc_body,
        grid=grid,
        in_specs=[
            row_spec, row_spec,
            rep((D, 2 * D)), rep((1, 2 * D)), rep((2 * D, C)), rep((1, C)),
            rep((1, D)), rep((1, D)), rep((1, C)), rep((1, C)),
            rep((C, 2 * C)), rep((1, 2 * C)), rep((2 * C, D)), rep((1, D)),
        ],
        out_specs=row_spec,
        out_shape=jax.ShapeDtypeStruct((n, D), jnp.float32),
    )(bg, sg, W1d, b1d.reshape(1, -1), W2d, b2d.reshape(1, -1),
      ln_d_g.reshape(1, -1), ln_d_b.reshape(1, -1),
      ln_u_g.reshape(1, -1), ln_u_b.reshape(1, -1),
      W1u, b1u.reshape(1, -1), W2u, b2u.reshape(1, -1))


def kernel(x, base_x, ln_d_g, ln_d_b, W1d, b1d, W2d, b2d,
           ln_u_g, ln_u_b, W1u, b1u, W2u, b2u, indices_subnodes):
    idx_flat = indices_subnodes.reshape(POS).astype(jnp.int32)
    bg, sg = _sc_gather_scatter(x, base_x, idx_flat)
    return _tc_mlp(bg, sg, W1d, b1d, W2d, b2d, ln_d_g, ln_d_b,
                   ln_u_g, ln_u_b, W1u, b1u, W2u, b2u)
